# Initial kernel scaffold; baseline (speedup 1.0000x reference)
import jax
import jax.numpy as jnp
from jax.experimental import pallas as pl


def kernel(x, adj, W1, b1, W2, b2):
    raise NotImplementedError("write your pallas kernel here")


# speedup vs baseline: 9.8101x; 9.8101x over previous
import functools

import jax
import jax.numpy as jnp
from jax import lax
from jax.experimental import pallas as pl
from jax.experimental.pallas import tpu as pltpu
from jax.experimental.pallas import tpu_sc as plsc

N = 10000
E = 320000
D = 128
NC = 2
NS = 16
C = 128
NW = NC * NS
K = 80
KH = K // 2
E_PAD = NW * C * K
N_PAD = 10112
N_TRASH = N_PAD - N
ROWS_PER_TILE = N_PAD // NS


def _agg_body(h_hbm, src_hbm, dst_hbm, out_hbm,
              acc, src_v, dst_v, rows0, rows1, sem0, sem1):
    c = lax.axis_index("c")
    s = lax.axis_index("s")
    wid = c * NS + s

    r0 = s * ROWS_PER_TILE

    def zrow(i, _):
        for j in range(D // 16):
            rows0[i, pl.ds(j * 16, 16)] = jnp.zeros((16,), jnp.float32)
        return 0

    lax.fori_loop(0, C, zrow, 0)
    off = 0
    while off < ROWS_PER_TILE:
        sz = min(C, ROWS_PER_TILE - off)
        pltpu.sync_copy(rows0.at[pl.ds(0, sz)], acc.at[pl.ds(r0 + off, sz)])
        off += sz
    plsc.subcore_barrier()

    def step(i, _):
        j = 2 * i
        d0 = pltpu.async_copy(h_hbm.at[src_v.at[j]], rows0, sem0)
        d1 = pltpu.async_copy(h_hbm.at[src_v.at[j + 1]], rows1, sem1)
        d0.wait()
        pltpu.sync_copy(rows0, acc.at[dst_v.at[j]], add=True)
        d1.wait()
        pltpu.sync_copy(rows1, acc.at[dst_v.at[j + 1]], add=True)
        return 0

    for h in range(2):
        pltpu.sync_copy(src_hbm.at[wid, pl.ds(h * KH, KH)], src_v)
        pltpu.sync_copy(dst_hbm.at[wid, pl.ds(h * KH, KH)], dst_v)
        lax.fori_loop(0, KH // 2, step, 0)
    plsc.subcore_barrier()
    pltpu.sync_copy(acc.at[pl.ds(r0, ROWS_PER_TILE)],
                    out_hbm.at[c, pl.ds(r0, ROWS_PER_TILE)])


_agg = pl.kernel(
    _agg_body,
    out_type=jax.ShapeDtypeStruct((NC, N_PAD, D), jnp.float32),
    mesh=plsc.VectorSubcoreMesh(core_axis_name="c", subcore_axis_name="s"),
    scratch_types=[
        pltpu.VMEM_SHARED((N_PAD, D), jnp.float32),
        pltpu.VMEM((KH, C), jnp.int32),
        pltpu.VMEM((KH, C), jnp.int32),
        pltpu.VMEM((C, D), jnp.float32),
        pltpu.VMEM((C, D), jnp.float32),
        pltpu.SemaphoreType.DMA,
        pltpu.SemaphoreType.DMA,
    ],
)


def _mm_body(p_ref, w_ref, b_ref, o_ref, *, relu):
    su = p_ref[0] + p_ref[1]
    o = jnp.dot(su, w_ref[...], preferred_element_type=jnp.float32) + b_ref[...]
    if relu:
        o = jnp.maximum(o, 0.0)
    o_ref[...] = o


def _mm(partials, w, b, relu):
    BM = 1000
    return pl.pallas_call(
        functools.partial(_mm_body, relu=relu),
        grid=(N // BM,),
        in_specs=[
            pl.BlockSpec((NC, BM, D), lambda i: (0, i, 0)),
            pl.BlockSpec((D, D), lambda i: (0, 0)),
            pl.BlockSpec((1, D), lambda i: (0, 0)),
        ],
        out_specs=pl.BlockSpec((BM, D), lambda i: (i, 0)),
        out_shape=jax.ShapeDtypeStruct((N, D), jnp.float32),
    )(partials, w, b.reshape(1, D))


def kernel(x, adj, W1, b1, W2, b2):
    src = adj[0].astype(jnp.int32)
    dst = adj[1].astype(jnp.int32)
    npad = E_PAD - E
    pad_src = jnp.arange(npad, dtype=jnp.int32) % N
    pad_dst = N + jnp.arange(npad, dtype=jnp.int32) % N_TRASH
    src_p = jnp.concatenate([src, pad_src]).reshape(NW, K, C)
    dst_p = jnp.concatenate([dst, pad_dst]).reshape(NW, K, C)

    p1 = _agg(x, src_p, dst_p)
    h1 = _mm(p1, W1, b1, relu=True)
    p2 = _agg(h1, src_p, dst_p)
    return _mm(p2, W2, b2, relu=False)
